# baseline (speedup 1.0000x reference)
import dataclasses
import functools

import jax
import jax.numpy as jnp
from jax import lax
from jax.experimental import pallas as pl
from jax.experimental.pallas import tpu as pltpu
from jax.experimental.pallas import tpu_sc as plsc

N_BINS = 15
NSLOTS = 16
LANES = 16
NW = 32
CHUNK = 16384

TC_BLOCK_ROWS = 8192
TC_ELEMS_PER_BLOCK = TC_BLOCK_ROWS * 128
TC_BLOCKS = 4


def _sc_hist(conf_hbm, acc_hbm, out_hbm, cnt_ref, csum_ref, asum_ref):
    zeros = jnp.zeros((LANES,), jnp.float32)
    for r in range(NSLOTS):
        cnt_ref[r, :] = zeros
        csum_ref[r, :] = zeros
        asum_ref[r, :] = zeros

    lanes = lax.iota(jnp.int32, LANES)
    ones = jnp.ones((LANES,), jnp.float32)

    def body(c_vm, a_vm):
        @plsc.parallel_loop(0, CHUNK, step=LANES, unroll=32)
        def _(j):
            c = c_vm[pl.ds(j, LANES)]
            a = a_vm[pl.ds(j, LANES)]
            b = (c * jnp.float32(N_BINS)).astype(jnp.int32)
            b = jnp.clip(b, 0, NSLOTS - 1)
            plsc.addupdate_scatter(cnt_ref, [b, lanes], ones)
            plsc.addupdate_scatter(csum_ref, [b, lanes], c)
            plsc.addupdate_scatter(asum_ref, [b, lanes], a)

    n = conf_hbm.shape[0]
    skip = TC_BLOCKS * TC_ELEMS_PER_BLOCK // CHUNK
    pltpu.emit_pipeline(
        body,
        grid=(n // CHUNK - skip,),
        in_specs=[
            pl.BlockSpec((CHUNK,), lambda i: (i + skip,)),
            pl.BlockSpec((CHUNK,), lambda i: (i + skip,)),
        ],
        out_specs=[],
        core_axis_name=("c", "s"),
        dimension_semantics=(pltpu.PARALLEL,),
    )(conf_hbm, acc_hbm)

    wid = lax.axis_index("c") * 16 + lax.axis_index("s")
    pltpu.sync_copy(cnt_ref, out_hbm.at[0, wid])
    pltpu.sync_copy(csum_ref, out_hbm.at[1, wid])
    pltpu.sync_copy(asum_ref, out_hbm.at[2, wid])


def _tc_hist_body(c_ref, a_ref, o_ref, acc_ref):
    step = pl.program_id(0)

    @pl.when(step == 0)
    def _init():
        acc_ref[...] = jnp.zeros_like(acc_ref)

    c = c_ref[...]
    a = a_ref[...]
    b = jnp.ceil(c * jnp.float32(N_BINS)) - jnp.float32(1.0)
    for k in range(N_BINS):
        mf = (b == jnp.float32(k)).astype(jnp.float32)
        acc_ref[k, :] += jnp.sum(mf, axis=0)
        acc_ref[k + 16, :] += jnp.sum(mf * c, axis=0)
        acc_ref[k + 32, :] += jnp.sum(mf * a, axis=0)

    @pl.when(step == pl.num_programs(0) - 1)
    def _finish():
        o_ref[...] = acc_ref[...]


def _finish_body(n_total, sc_ref, tc_ref, o_ref):
    sc = sc_ref[...]
    tot = jnp.sum(sc, axis=(1, 3))
    tc = jnp.reshape(tc_ref[...], (3, 16, 128))
    tot = tot + jnp.sum(tc, axis=2)
    cnt = tot[0:1, 0:N_BINS]
    csum = tot[1:2, 0:N_BINS]
    asum = tot[2:3, 0:N_BINS]
    safe = jnp.maximum(cnt, 1.0)
    diff = (csum - asum) / safe
    contrib = diff * diff * (cnt / jnp.float32(n_total))
    contrib = jnp.where(cnt > 0, contrib, 0.0)
    o_ref[...] = jnp.sum(contrib, axis=(0, 1), keepdims=True)


def kernel(confidences, accuracies):
    n = confidences.shape[0]
    mesh = plsc.VectorSubcoreMesh(core_axis_name="c", subcore_axis_name="s")
    cp = pltpu.CompilerParams()
    if "needs_layout_passes" in pltpu.CompilerParams.__dataclass_fields__:
        cp = dataclasses.replace(cp, needs_layout_passes=False)
    hist_sc = pl.kernel(
        _sc_hist,
        out_type=jax.ShapeDtypeStruct((3, NW, NSLOTS, LANES), jnp.float32),
        mesh=mesh,
        scratch_types=[
            pltpu.VMEM((NSLOTS, LANES), jnp.float32),
            pltpu.VMEM((NSLOTS, LANES), jnp.float32),
            pltpu.VMEM((NSLOTS, LANES), jnp.float32),
        ],
        compiler_params=cp,
    )(confidences, accuracies)

    part_tc = pl.pallas_call(
        _tc_hist_body,
        grid=(TC_BLOCKS,),
        in_specs=[
            pl.BlockSpec((TC_BLOCK_ROWS, 128), lambda i: (i, 0)),
            pl.BlockSpec((TC_BLOCK_ROWS, 128), lambda i: (i, 0)),
        ],
        out_specs=pl.BlockSpec((48, 128), lambda i: (0, 0)),
        out_shape=jax.ShapeDtypeStruct((48, 128), jnp.float32),
        scratch_shapes=[pltpu.VMEM((48, 128), jnp.float32)],
    )(
        confidences.reshape(n // 128, 128),
        accuracies.reshape(n // 128, 128),
    )

    out = pl.pallas_call(
        functools.partial(_finish_body, n),
        out_shape=jax.ShapeDtypeStruct((1, 1), jnp.float32),
    )(hist_sc, part_tc)
    return out[0, 0]


# speedup vs baseline: 1.6484x; 1.6484x over previous
import dataclasses
import functools

import jax
import jax.numpy as jnp
from jax import lax
from jax.experimental import pallas as pl
from jax.experimental.pallas import tpu as pltpu
from jax.experimental.pallas import tpu_sc as plsc

N_BINS = 15
NSLOTS = 16
LANES = 16
NW = 32
CHUNK = 16384

TC_BLOCK_ROWS = 8192
TC_ELEMS_PER_BLOCK = TC_BLOCK_ROWS * 128
TC_BLOCKS = 5


def _sc_hist(conf_hbm, acc_hbm, out_hbm, cnt_ref, csum_ref, asum_ref):
    zeros = jnp.zeros((LANES,), jnp.float32)
    for r in range(NSLOTS):
        cnt_ref[r, :] = zeros
        csum_ref[r, :] = zeros
        asum_ref[r, :] = zeros

    lanes = lax.iota(jnp.int32, LANES)
    ones = jnp.ones((LANES,), jnp.float32)

    def body(c_vm, a_vm):
        @plsc.parallel_loop(0, CHUNK, step=LANES, unroll=16)
        def _(j):
            c = c_vm[pl.ds(j, LANES)]
            a = a_vm[pl.ds(j, LANES)]
            b = (c * jnp.float32(N_BINS)).astype(jnp.int32)
            b = jnp.clip(b, 0, NSLOTS - 1)
            plsc.addupdate_scatter(cnt_ref, [b, lanes], ones)
            plsc.addupdate_scatter(csum_ref, [b, lanes], c)
            plsc.addupdate_scatter(asum_ref, [b, lanes], a)

    n = conf_hbm.shape[0]
    skip = TC_BLOCKS * TC_ELEMS_PER_BLOCK // CHUNK
    pltpu.emit_pipeline(
        body,
        grid=(n // CHUNK - skip,),
        in_specs=[
            pl.BlockSpec((CHUNK,), lambda i: (i + skip,)),
            pl.BlockSpec((CHUNK,), lambda i: (i + skip,)),
        ],
        out_specs=[],
        core_axis_name=("c", "s"),
        dimension_semantics=(pltpu.PARALLEL,),
    )(conf_hbm, acc_hbm)

    wid = lax.axis_index("c") * 16 + lax.axis_index("s")
    pltpu.sync_copy(cnt_ref, out_hbm.at[0, wid])
    pltpu.sync_copy(csum_ref, out_hbm.at[1, wid])
    pltpu.sync_copy(asum_ref, out_hbm.at[2, wid])


def _tc_hist_body(c_ref, a_ref, o_ref, acc_ref):
    step = pl.program_id(0)

    @pl.when(step == 0)
    def _init():
        acc_ref[...] = jnp.zeros_like(acc_ref)

    c = c_ref[...]
    a = a_ref[...]
    b = jnp.ceil(c * jnp.float32(N_BINS)) - jnp.float32(1.0)
    for k in range(N_BINS):
        mf = (b == jnp.float32(k)).astype(jnp.float32)
        acc_ref[k, :] += jnp.sum(mf, axis=0)
        acc_ref[k + 16, :] += jnp.sum(mf * c, axis=0)
        acc_ref[k + 32, :] += jnp.sum(mf * a, axis=0)

    @pl.when(step == pl.num_programs(0) - 1)
    def _finish():
        o_ref[...] = acc_ref[...]


def _finish_body(n_total, sc_ref, tc_ref, o_ref):
    sc = sc_ref[...]
    tot = jnp.sum(sc, axis=(1, 3))
    tc = jnp.reshape(tc_ref[...], (3, 16, 128))
    tot = tot + jnp.sum(tc, axis=2)
    cnt = tot[0:1, 0:N_BINS]
    csum = tot[1:2, 0:N_BINS]
    asum = tot[2:3, 0:N_BINS]
    safe = jnp.maximum(cnt, 1.0)
    diff = (csum - asum) / safe
    contrib = diff * diff * (cnt / jnp.float32(n_total))
    contrib = jnp.where(cnt > 0, contrib, 0.0)
    o_ref[...] = jnp.sum(contrib, axis=(0, 1), keepdims=True)


def kernel(confidences, accuracies):
    n = confidences.shape[0]
    mesh = plsc.VectorSubcoreMesh(core_axis_name="c", subcore_axis_name="s")
    cp = pltpu.CompilerParams()
    if "needs_layout_passes" in pltpu.CompilerParams.__dataclass_fields__:
        cp = dataclasses.replace(cp, needs_layout_passes=False)
    hist_sc = pl.kernel(
        _sc_hist,
        out_type=jax.ShapeDtypeStruct((3, NW, NSLOTS, LANES), jnp.float32),
        mesh=mesh,
        scratch_types=[
            pltpu.VMEM((NSLOTS, LANES), jnp.float32),
            pltpu.VMEM((NSLOTS, LANES), jnp.float32),
            pltpu.VMEM((NSLOTS, LANES), jnp.float32),
        ],
        compiler_params=cp,
    )(confidences, accuracies)

    part_tc = pl.pallas_call(
        _tc_hist_body,
        grid=(TC_BLOCKS,),
        in_specs=[
            pl.BlockSpec((TC_BLOCK_ROWS, 128), lambda i: (i, 0)),
            pl.BlockSpec((TC_BLOCK_ROWS, 128), lambda i: (i, 0)),
        ],
        out_specs=pl.BlockSpec((48, 128), lambda i: (0, 0)),
        out_shape=jax.ShapeDtypeStruct((48, 128), jnp.float32),
        scratch_shapes=[pltpu.VMEM((48, 128), jnp.float32)],
    )(
        confidences.reshape(n // 128, 128),
        accuracies.reshape(n // 128, 128),
    )

    out = pl.pallas_call(
        functools.partial(_finish_body, n),
        out_shape=jax.ShapeDtypeStruct((1, 1), jnp.float32),
    )(hist_sc, part_tc)
    return out[0, 0]


# speedup vs baseline: 1.9989x; 1.2126x over previous
import dataclasses
import functools

import jax
import jax.numpy as jnp
from jax import lax
from jax.experimental import pallas as pl
from jax.experimental.pallas import tpu as pltpu
from jax.experimental.pallas import tpu_sc as plsc

N_BINS = 15
NSLOTS = 16
LANES = 16
NW = 32
CHUNK = 16384

TC_BLOCK_ROWS = 8192
TC_ELEMS_PER_BLOCK = TC_BLOCK_ROWS * 128
TC_BLOCKS = 4


def _sc_hist(conf_hbm, acc_hbm, out_hbm, cnt_ref, dsum_ref):
    zeros = jnp.zeros((LANES,), jnp.float32)
    for r in range(NSLOTS):
        cnt_ref[r, :] = zeros
        dsum_ref[r, :] = zeros

    lanes = lax.iota(jnp.int32, LANES)
    ones = jnp.ones((LANES,), jnp.float32)

    def body(c_vm, a_vm):
        @plsc.parallel_loop(0, CHUNK, step=LANES, unroll=16)
        def _(j):
            c = c_vm[pl.ds(j, LANES)]
            a = a_vm[pl.ds(j, LANES)]
            b = (c * jnp.float32(N_BINS)).astype(jnp.int32)
            b = jnp.clip(b, 0, NSLOTS - 1)
            plsc.addupdate_scatter(cnt_ref, [b, lanes], ones)
            plsc.addupdate_scatter(dsum_ref, [b, lanes], c - a)

    n = conf_hbm.shape[0]
    skip = TC_BLOCKS * TC_ELEMS_PER_BLOCK // CHUNK
    pltpu.emit_pipeline(
        body,
        grid=(n // CHUNK - skip,),
        in_specs=[
            pl.BlockSpec((CHUNK,), lambda i: (i + skip,)),
            pl.BlockSpec((CHUNK,), lambda i: (i + skip,)),
        ],
        out_specs=[],
        core_axis_name=("c", "s"),
        dimension_semantics=(pltpu.PARALLEL,),
    )(conf_hbm, acc_hbm)

    wid = lax.axis_index("c") * 16 + lax.axis_index("s")
    pltpu.sync_copy(cnt_ref, out_hbm.at[0, wid])
    pltpu.sync_copy(dsum_ref, out_hbm.at[1, wid])


def _tc_hist_body(c_ref, a_ref, o_ref, acc_ref):
    step = pl.program_id(0)

    @pl.when(step == 0)
    def _init():
        acc_ref[...] = jnp.zeros_like(acc_ref)

    c = c_ref[...]
    a = a_ref[...]
    b = jnp.ceil(c * jnp.float32(N_BINS)) - jnp.float32(1.0)
    d = c - a
    for k in range(N_BINS):
        mf = (b == jnp.float32(k)).astype(jnp.float32)
        acc_ref[k, :] += jnp.sum(mf, axis=0)
        acc_ref[k + 16, :] += jnp.sum(mf * d, axis=0)

    @pl.when(step == pl.num_programs(0) - 1)
    def _finish():
        o_ref[...] = acc_ref[...]


def _finish_body(n_total, sc_ref, tc_ref, o_ref):
    sc = sc_ref[...]
    tot = jnp.sum(sc, axis=(1, 3))
    tc = jnp.reshape(tc_ref[...], (2, 16, 128))
    tot = tot + jnp.sum(tc, axis=2)
    cnt = tot[0:1, 0:N_BINS]
    dsum = tot[1:2, 0:N_BINS]
    safe = jnp.maximum(cnt, 1.0)
    diff = dsum / safe
    contrib = diff * diff * (cnt / jnp.float32(n_total))
    contrib = jnp.where(cnt > 0, contrib, 0.0)
    o_ref[...] = jnp.sum(contrib, axis=(0, 1), keepdims=True)


def kernel(confidences, accuracies):
    n = confidences.shape[0]
    mesh = plsc.VectorSubcoreMesh(core_axis_name="c", subcore_axis_name="s")
    cp = pltpu.CompilerParams()
    if "needs_layout_passes" in pltpu.CompilerParams.__dataclass_fields__:
        cp = dataclasses.replace(cp, needs_layout_passes=False)
    hist_sc = pl.kernel(
        _sc_hist,
        out_type=jax.ShapeDtypeStruct((2, NW, NSLOTS, LANES), jnp.float32),
        mesh=mesh,
        scratch_types=[
            pltpu.VMEM((NSLOTS, LANES), jnp.float32),
            pltpu.VMEM((NSLOTS, LANES), jnp.float32),
        ],
        compiler_params=cp,
    )(confidences, accuracies)

    part_tc = pl.pallas_call(
        _tc_hist_body,
        grid=(TC_BLOCKS,),
        in_specs=[
            pl.BlockSpec((TC_BLOCK_ROWS, 128), lambda i: (i, 0)),
            pl.BlockSpec((TC_BLOCK_ROWS, 128), lambda i: (i, 0)),
        ],
        out_specs=pl.BlockSpec((32, 128), lambda i: (0, 0)),
        out_shape=jax.ShapeDtypeStruct((32, 128), jnp.float32),
        scratch_shapes=[pltpu.VMEM((32, 128), jnp.float32)],
    )(
        confidences.reshape(n // 128, 128),
        accuracies.reshape(n // 128, 128),
    )

    out = pl.pallas_call(
        functools.partial(_finish_body, n),
        out_shape=jax.ShapeDtypeStruct((1, 1), jnp.float32),
    )(hist_sc, part_tc)
    return out[0, 0]


# speedup vs baseline: 2.1193x; 1.0602x over previous
import dataclasses
import functools

import jax
import jax.numpy as jnp
from jax import lax
from jax.experimental import pallas as pl
from jax.experimental.pallas import tpu as pltpu
from jax.experimental.pallas import tpu_sc as plsc

N_BINS = 15
NSLOTS = 16
LANES = 16
NW = 32
CHUNK = 16384

TC_BLOCK_ROWS = 8192
TC_ELEMS_PER_BLOCK = TC_BLOCK_ROWS * 128
TC_BLOCKS = 5


def _sc_hist(conf_hbm, acc_hbm, out_hbm, cnt_ref, dsum_ref):
    zeros = jnp.zeros((LANES,), jnp.float32)
    for r in range(NSLOTS):
        cnt_ref[r, :] = zeros
        dsum_ref[r, :] = zeros

    lanes = lax.iota(jnp.int32, LANES)
    ones = jnp.ones((LANES,), jnp.float32)

    def body(c_vm, a_vm):
        @plsc.parallel_loop(0, CHUNK, step=LANES, unroll=16)
        def _(j):
            c = c_vm[pl.ds(j, LANES)]
            a = a_vm[pl.ds(j, LANES)]
            b = (c * jnp.float32(N_BINS)).astype(jnp.int32)
            b = jnp.clip(b, 0, NSLOTS - 1)
            plsc.addupdate_scatter(cnt_ref, [b, lanes], ones)
            plsc.addupdate_scatter(dsum_ref, [b, lanes], c - a)

    n = conf_hbm.shape[0]
    skip = TC_BLOCKS * TC_ELEMS_PER_BLOCK // CHUNK
    pltpu.emit_pipeline(
        body,
        grid=(n // CHUNK - skip,),
        in_specs=[
            pl.BlockSpec((CHUNK,), lambda i: (i + skip,)),
            pl.BlockSpec((CHUNK,), lambda i: (i + skip,)),
        ],
        out_specs=[],
        core_axis_name=("c", "s"),
        dimension_semantics=(pltpu.PARALLEL,),
    )(conf_hbm, acc_hbm)

    wid = lax.axis_index("c") * 16 + lax.axis_index("s")
    pltpu.sync_copy(cnt_ref, out_hbm.at[0, wid])
    pltpu.sync_copy(dsum_ref, out_hbm.at[1, wid])


def _tc_hist_body(c_ref, a_ref, o_ref, acc_ref):
    step = pl.program_id(0)

    @pl.when(step == 0)
    def _init():
        acc_ref[...] = jnp.zeros_like(acc_ref)

    c = c_ref[...]
    a = a_ref[...]
    b = jnp.ceil(c * jnp.float32(N_BINS)) - jnp.float32(1.0)
    d = c - a
    for k in range(N_BINS):
        mf = (b == jnp.float32(k)).astype(jnp.float32)
        acc_ref[k, :] += jnp.sum(mf, axis=0)
        acc_ref[k + 16, :] += jnp.sum(mf * d, axis=0)

    @pl.when(step == pl.num_programs(0) - 1)
    def _finish():
        o_ref[...] = acc_ref[...]


def _finish_body(n_total, sc_ref, tc_ref, o_ref):
    sc = sc_ref[...]
    tot = jnp.sum(sc, axis=(1, 3))
    tc = jnp.reshape(tc_ref[...], (2, 16, 128))
    tot = tot + jnp.sum(tc, axis=2)
    cnt = tot[0:1, 0:N_BINS]
    dsum = tot[1:2, 0:N_BINS]
    safe = jnp.maximum(cnt, 1.0)
    diff = dsum / safe
    contrib = diff * diff * (cnt / jnp.float32(n_total))
    contrib = jnp.where(cnt > 0, contrib, 0.0)
    o_ref[...] = jnp.sum(contrib, axis=(0, 1), keepdims=True)


def kernel(confidences, accuracies):
    n = confidences.shape[0]
    mesh = plsc.VectorSubcoreMesh(core_axis_name="c", subcore_axis_name="s")
    cp = pltpu.CompilerParams()
    if "needs_layout_passes" in pltpu.CompilerParams.__dataclass_fields__:
        cp = dataclasses.replace(cp, needs_layout_passes=False)
    hist_sc = pl.kernel(
        _sc_hist,
        out_type=jax.ShapeDtypeStruct((2, NW, NSLOTS, LANES), jnp.float32),
        mesh=mesh,
        scratch_types=[
            pltpu.VMEM((NSLOTS, LANES), jnp.float32),
            pltpu.VMEM((NSLOTS, LANES), jnp.float32),
        ],
        compiler_params=cp,
    )(confidences, accuracies)

    part_tc = pl.pallas_call(
        _tc_hist_body,
        grid=(TC_BLOCKS,),
        in_specs=[
            pl.BlockSpec((TC_BLOCK_ROWS, 128), lambda i: (i, 0)),
            pl.BlockSpec((TC_BLOCK_ROWS, 128), lambda i: (i, 0)),
        ],
        out_specs=pl.BlockSpec((32, 128), lambda i: (0, 0)),
        out_shape=jax.ShapeDtypeStruct((32, 128), jnp.float32),
        scratch_shapes=[pltpu.VMEM((32, 128), jnp.float32)],
    )(
        confidences.reshape(n // 128, 128),
        accuracies.reshape(n // 128, 128),
    )

    out = pl.pallas_call(
        functools.partial(_finish_body, n),
        out_shape=jax.ShapeDtypeStruct((1, 1), jnp.float32),
    )(hist_sc, part_tc)
    return out[0, 0]
